# baseline (speedup 1.0000x reference)
import jax
import jax.numpy as jnp
from jax.experimental import pallas as pl
from jax.experimental.pallas import tpu as pltpu

_N_BLK = 512
_N_BULK_BLOCKS = 195
_TAIL = 160


def _mm_body(x_ref, m_ref, o_ref):
    o_ref[...] = jax.lax.dot_general(
        x_ref[...].astype(jnp.bfloat16),
        m_ref[...].astype(jnp.bfloat16),
        dimension_numbers=(((1,), (1,)), ((), ())),
        preferred_element_type=jnp.float32,
    )


def kernel(x, target, content, memory):
    b, f = x.shape
    n = memory.shape[0]
    mesh = pltpu.create_tensorcore_mesh("core")

    @pl.kernel(out_type=jax.ShapeDtypeStruct((b, n), jnp.float32), mesh=mesh)
    def run(x_hbm, m_hbm, o_hbm):
        pltpu.emit_pipeline(
            _mm_body,
            grid=(_N_BULK_BLOCKS,),
            in_specs=[
                pl.BlockSpec((b, f), lambda i: (0, 0)),
                pl.BlockSpec((_N_BLK, f), lambda i: (i, 0)),
            ],
            out_specs=[pl.BlockSpec((b, _N_BLK), lambda i: (0, i))],
            core_axis_name="core",
        )(x_hbm, m_hbm, o_hbm)
        pltpu.emit_pipeline(
            _mm_body,
            grid=(1,),
            in_specs=[
                pl.BlockSpec((b, f), lambda i: (0, 0)),
                pl.BlockSpec((_TAIL, f), lambda i: (_N_BULK_BLOCKS * _N_BLK // _TAIL, 0)),
            ],
            out_specs=[pl.BlockSpec((b, _TAIL), lambda i: (0, _N_BULK_BLOCKS * _N_BLK // _TAIL))],
            core_axis_name="core",
        )(x_hbm, m_hbm, o_hbm)

    return run(x, memory)
